# Initial kernel scaffold; baseline (speedup 1.0000x reference)
import functools

import jax
import jax.numpy as jnp
from jax import lax
from jax.experimental import pallas as pl
from jax.experimental.pallas import tpu as pltpu
from jax.experimental.pallas import tpu_sc as plsc

BATCH = 16384
FIELDS = 100
EMBED_DIM = 32
NROWS = BATCH * FIELDS

NUM_CORES = 2
NUM_SUBCORES = 16
NUM_WORKERS = NUM_CORES * NUM_SUBCORES
ROWS_PER_WORKER = NROWS // NUM_WORKERS
CHUNK = 2048
NUM_CHUNKS = ROWS_PER_WORKER // CHUNK


def _make_gather():
    mesh = plsc.VectorSubcoreMesh(core_axis_name="c", subcore_axis_name="s")

    @functools.partial(
        pl.kernel,
        mesh=mesh,
        out_type=jax.ShapeDtypeStruct((NROWS, EMBED_DIM), jnp.float32),
        scratch_types=[
            pltpu.VMEM((CHUNK,), jnp.int32),
            pltpu.VMEM((CHUNK, EMBED_DIM), jnp.float32),
            pltpu.SemaphoreType.DMA,
        ],
    )
    def gather_kernel(idx_hbm, table_hbm, out_hbm, idx_v, rows_v, sem):
        wid = lax.axis_index("s") * NUM_CORES + lax.axis_index("c")
        base = wid * ROWS_PER_WORKER

        def body(g, carry):
            off = base + g * CHUNK
            pltpu.sync_copy(idx_hbm.at[pl.ds(off, CHUNK)], idx_v)
            pltpu.async_copy(table_hbm.at[idx_v], rows_v, sem).wait()
            pltpu.sync_copy(rows_v, out_hbm.at[pl.ds(off, CHUNK)])
            return carry

        lax.fori_loop(0, NUM_CHUNKS, body, 0)

    return gather_kernel


_gather = _make_gather()


def kernel(indices, table):
    flat = indices.reshape(NROWS)
    out = _gather(flat, table)
    return out.reshape(BATCH, FIELDS, EMBED_DIM)


# speedup vs baseline: 1.1077x; 1.1077x over previous
import functools

import jax
import jax.numpy as jnp
from jax import lax
from jax.experimental import pallas as pl
from jax.experimental.pallas import tpu as pltpu
from jax.experimental.pallas import tpu_sc as plsc

BATCH = 16384
FIELDS = 100
EMBED_DIM = 32
NROWS = BATCH * FIELDS

NUM_CORES = 2
NUM_SUBCORES = 16
NUM_WORKERS = NUM_CORES * NUM_SUBCORES
ROWS_PER_WORKER = NROWS // NUM_WORKERS
CHUNK = 2048
NUM_CHUNKS = ROWS_PER_WORKER // CHUNK


def _make_gather():
    mesh = plsc.VectorSubcoreMesh(core_axis_name="c", subcore_axis_name="s")

    @functools.partial(
        pl.kernel,
        mesh=mesh,
        out_type=jax.ShapeDtypeStruct((NROWS, EMBED_DIM), jnp.float32),
        scratch_types=[
            pltpu.VMEM((CHUNK,), jnp.int32),
            pltpu.VMEM((CHUNK, EMBED_DIM), jnp.float32),
            pltpu.SemaphoreType.DMA,
        ],
        compiler_params=pltpu.CompilerParams(use_tc_tiling_on_sc=False),
    )
    def gather_kernel(idx_hbm, table_hbm, out_hbm, idx_v, rows_v, sem):
        wid = lax.axis_index("s") * NUM_CORES + lax.axis_index("c")
        base = wid * ROWS_PER_WORKER

        def body(g, carry):
            off = base + g * CHUNK
            pltpu.sync_copy(idx_hbm.at[pl.ds(off, CHUNK)], idx_v)
            pltpu.async_copy(table_hbm.at[idx_v], rows_v, sem).wait()
            pltpu.sync_copy(rows_v, out_hbm.at[pl.ds(off, CHUNK)])
            return carry

        lax.fori_loop(0, NUM_CHUNKS, body, 0)

    return gather_kernel


_gather = _make_gather()


def kernel(indices, table):
    flat = indices.reshape(NROWS)
    out = _gather(flat, table)
    return out.reshape(BATCH, FIELDS, EMBED_DIM)


# speedup vs baseline: 4.4218x; 3.9919x over previous
import functools

import jax
import jax.numpy as jnp
from jax import lax
from jax.experimental import pallas as pl
from jax.experimental.pallas import tpu as pltpu
from jax.experimental.pallas import tpu_sc as plsc

BATCH = 16384
FIELDS = 100
EMBED_DIM = 32

NUM_CORES = 2
NUM_SUBCORES = 16
NUM_WORKERS = NUM_CORES * NUM_SUBCORES
ROWS_PER_WORKER = BATCH // NUM_WORKERS
NB = 16
NUM_CHUNKS = ROWS_PER_WORKER // NB
NUM_PAIRS = NUM_CHUNKS // 2


def _make_gather():
    mesh = plsc.VectorSubcoreMesh(core_axis_name="c", subcore_axis_name="s")

    @functools.partial(
        pl.kernel,
        mesh=mesh,
        out_type=jax.ShapeDtypeStruct((BATCH, FIELDS, EMBED_DIM), jnp.float32),
        scratch_types=[
            pltpu.VMEM((2, NB, FIELDS), jnp.int32),
            pltpu.VMEM((2, NB, FIELDS, EMBED_DIM), jnp.float32),
            pltpu.SemaphoreType.DMA,
            pltpu.SemaphoreType.DMA,
            pltpu.SemaphoreType.DMA,
            pltpu.SemaphoreType.DMA,
            pltpu.SemaphoreType.DMA,
            pltpu.SemaphoreType.DMA,
        ],
        compiler_params=pltpu.CompilerParams(use_tc_tiling_on_sc=False),
    )
    def gather_kernel(idx_hbm, table_hbm, out_hbm, idx_v, rows_v,
                      idx_sem0, idx_sem1, gat_sem0, gat_sem1,
                      st_sem0, st_sem1):
        wid = lax.axis_index("s") * NUM_CORES + lax.axis_index("c")
        base = wid * ROWS_PER_WORKER
        idx_sems = (idx_sem0, idx_sem1)
        gat_sems = (gat_sem0, gat_sem1)
        st_sems = (st_sem0, st_sem1)

        for s in (0, 1):
            pltpu.async_copy(
                idx_hbm.at[pl.ds(base + s * NB, NB), :], idx_v.at[s],
                idx_sems[s])

        def body(gpair, carry):
            for s in (0, 1):
                g = gpair * 2 + s
                b0 = base + g * NB

                pltpu.make_async_copy(
                    idx_hbm.at[pl.ds(b0, NB), :], idx_v.at[s],
                    idx_sems[s]).wait()

                @pl.when(gpair >= 1)
                def _():
                    pltpu.make_async_copy(
                        rows_v.at[s], out_hbm.at[pl.ds(b0 - 2 * NB, NB)],
                        st_sems[s]).wait()

                copies = [
                    pltpu.async_copy(
                        table_hbm.at[idx_v.at[s, j]], rows_v.at[s, j],
                        gat_sems[s])
                    for j in range(NB)
                ]
                for c in copies:
                    c.wait()

                pltpu.async_copy(
                    rows_v.at[s], out_hbm.at[pl.ds(b0, NB)], st_sems[s])

                @pl.when(gpair < NUM_PAIRS - 1)
                def _():
                    pltpu.async_copy(
                        idx_hbm.at[pl.ds(b0 + 2 * NB, NB), :], idx_v.at[s],
                        idx_sems[s])

            return carry

        lax.fori_loop(0, NUM_PAIRS, body, 0)

        for s in (0, 1):
            g = NUM_CHUNKS - 2 + s
            pltpu.make_async_copy(
                rows_v.at[s], out_hbm.at[pl.ds(base + g * NB, NB)],
                st_sems[s]).wait()

    return gather_kernel


_gather = _make_gather()


def kernel(indices, table):
    return _gather(indices, table)
